# baseline (speedup 1.0000x reference)
import jax
import jax.numpy as jnp
from jax import lax
from jax.experimental import pallas as pl
from jax.experimental.pallas import tpu as pltpu
from jax.experimental.pallas import tpu_sc as plsc

L = 16
NC = 2
NS = 16
NW = NC * NS
ROWLEN = 64
NROWS_OUT = 16384
TOTAL = NROWS_OUT * ROWLEN
E = TOTAL // NW
GROUP = 128
NG = E // GROUP


def _body(x_hbm, idx_hbm, out_hbm, fidx_v, out_v, sem):
    wid = lax.axis_index("s") * NC + lax.axis_index("c")
    base = wid * E
    pltpu.sync_copy(idx_hbm.at[pl.ds(base, E)], fidx_v)

    col0 = lax.iota(jnp.int32, L)

    def compute(g, carry):
        for c in range(ROWLEN // L):
            off = g * ROWLEN + c * L
            v = fidx_v[pl.ds(off, L)]
            fidx_v[pl.ds(off, L)] = v * ROWLEN + (col0 + c * L)
        return carry

    lax.fori_loop(0, E // ROWLEN, compute, 0)

    def fire(r, carry):
        pltpu.async_copy(
            x_hbm.at[fidx_v.at[pl.ds(r * GROUP, GROUP)]],
            out_v.at[pl.ds(r * GROUP, GROUP)],
            sem,
        )
        return carry

    lax.fori_loop(0, NG, fire, 0)

    pltpu.make_async_copy(x_hbm.at[pl.ds(0, E)], out_v, sem).wait()

    pltpu.sync_copy(out_v, out_hbm.at[pl.ds(base, E)])


def kernel(x, dim, index):
    del dim
    xf = x.reshape(-1)
    idxf = index.astype(jnp.int32).reshape(-1)
    out = pl.kernel(
        _body,
        out_type=jax.ShapeDtypeStruct((TOTAL,), jnp.float32),
        mesh=plsc.VectorSubcoreMesh(core_axis_name="c", subcore_axis_name="s"),
        scratch_types=[
            pltpu.VMEM((E,), jnp.int32),
            pltpu.VMEM((E,), jnp.float32),
            pltpu.SemaphoreType.DMA,
        ],
    )(xf, idxf)
    return out.reshape(NROWS_OUT, ROWLEN)
